# baseline (speedup 1.0000x reference)
import functools

import jax
import jax.numpy as jnp
from jax import lax
from jax.experimental import pallas as pl
from jax.experimental.pallas import tpu as pltpu
from jax.experimental.pallas import tpu_sc as plsc

N = 10000
E = 320000
IN = 128
OUT = 128
H = 4
D = 32
NEG = 0.1

FX = 144
ERT = 16

NC = 2
NS = 16
NW = NC * NS
EW = E // NW
B = 80
CH = EW // B
G = B // 16
NPS = N // NS

_f32 = jnp.float32
_i32 = jnp.int32


def _dot(a, b):
    return jnp.dot(a, b, preferred_element_type=_f32,
                   precision=lax.Precision.HIGHEST)


def _layer1_body(x_ref, w1t_ref, ael_ref, aer_ref, featx_ref, ert_ref):
    f = _dot(x_ref[...], w1t_ref[...])
    elp = _dot(f, ael_ref[...])
    featx_ref[...] = jnp.concatenate([f, elp], axis=-1)
    ert_ref[...] = _dot(f, aer_ref[...])


def _layer2_body(acc_ref, w2t_ref, ael_ref, aer_ref, e16_ref,
                 featx_ref, ert_ref, rst1_ref):
    a = acc_ref[0] + acc_ref[1]
    den = _dot(a[:, 128:144], e16_ref[...]) + 1e-9
    r1 = jnp.maximum(a[:, :128] / den, 0.0)
    rst1_ref[...] = r1
    f = _dot(r1, w2t_ref[...])
    elp = _dot(f, ael_ref[...])
    featx_ref[...] = jnp.concatenate([f, elp], axis=-1)
    ert_ref[...] = _dot(f, aer_ref[...])


def _final_body(acc_ref, rst1_ref, wo1t_ref, wo2t_ref, e16_ref, b_ref,
                out_ref):
    a = acc_ref[0] + acc_ref[1]
    den = _dot(a[:, 128:144], e16_ref[...]) + 1e-9
    r2 = jnp.maximum(a[:, :128] / den, 0.0)
    out_ref[...] = (_dot(rst1_ref[...], wo1t_ref[...])
                    + _dot(r2, wo2t_ref[...]) + b_ref[...])


_R = 2000


def _tc_layer1(x, w1t, ael, aer):
    return pl.pallas_call(
        _layer1_body,
        grid=(N // _R,),
        in_specs=[
            pl.BlockSpec((_R, IN), lambda i: (i, 0)),
            pl.BlockSpec((IN, IN), lambda i: (0, 0)),
            pl.BlockSpec((IN, 16), lambda i: (0, 0)),
            pl.BlockSpec((IN, 16), lambda i: (0, 0)),
        ],
        out_specs=[
            pl.BlockSpec((_R, FX), lambda i: (i, 0)),
            pl.BlockSpec((_R, ERT), lambda i: (i, 0)),
        ],
        out_shape=[
            jax.ShapeDtypeStruct((N, FX), _f32),
            jax.ShapeDtypeStruct((N, ERT), _f32),
        ],
    )(x, w1t, ael, aer)


def _tc_layer2(acc, w2t, ael, aer, e16):
    return pl.pallas_call(
        _layer2_body,
        grid=(N // _R,),
        in_specs=[
            pl.BlockSpec((NC, _R, FX), lambda i: (0, i, 0)),
            pl.BlockSpec((IN, IN), lambda i: (0, 0)),
            pl.BlockSpec((IN, 16), lambda i: (0, 0)),
            pl.BlockSpec((IN, 16), lambda i: (0, 0)),
            pl.BlockSpec((16, IN), lambda i: (0, 0)),
        ],
        out_specs=[
            pl.BlockSpec((_R, FX), lambda i: (i, 0)),
            pl.BlockSpec((_R, ERT), lambda i: (i, 0)),
            pl.BlockSpec((_R, IN), lambda i: (i, 0)),
        ],
        out_shape=[
            jax.ShapeDtypeStruct((N, FX), _f32),
            jax.ShapeDtypeStruct((N, ERT), _f32),
            jax.ShapeDtypeStruct((N, IN), _f32),
        ],
    )(acc, w2t, ael, aer, e16)


def _tc_final(acc, rst1, wo1t, wo2t, e16, b):
    return pl.pallas_call(
        _final_body,
        grid=(N // _R,),
        in_specs=[
            pl.BlockSpec((NC, _R, FX), lambda i: (0, i, 0)),
            pl.BlockSpec((_R, IN), lambda i: (i, 0)),
            pl.BlockSpec((IN, OUT), lambda i: (0, 0)),
            pl.BlockSpec((IN, OUT), lambda i: (0, 0)),
            pl.BlockSpec((16, IN), lambda i: (0, 0)),
            pl.BlockSpec((1, OUT), lambda i: (0, 0)),
        ],
        out_specs=pl.BlockSpec((_R, OUT), lambda i: (i, 0)),
        out_shape=jax.ShapeDtypeStruct((N, OUT), _f32),
    )(acc, rst1, wo1t, wo2t, e16, b)


def _edge_body(featx_hbm, ert_hbm, srcr_hbm, dstr_hbm, ewr_hbm, out_hbm,
               src_v, dst_v, ew_v, rows_v, er_v, acc_sh, sem1, sem2, sem3):
    cid = lax.axis_index("c")
    sid = lax.axis_index("s")
    wid = sid * NC + cid

    pltpu.sync_copy(srcr_hbm.at[wid], src_v)
    pltpu.sync_copy(dstr_hbm.at[wid], dst_v)

    zv = jnp.zeros((16,), _f32)

    def zrow(r, c):
        for k in range(FX // 16):
            rows_v[r, pl.ds(k * 16, 16)] = zv
        return c

    lax.fori_loop(0, B, zrow, 0)

    def zblk(i, c):
        j = sid + NS * i

        @pl.when(j < N // B)
        def _():
            off = pl.multiple_of(j * B, 8)
            pltpu.sync_copy(rows_v, acc_sh.at[pl.ds(off, B)])
        return c

    lax.fori_loop(0, (N // B + NS - 1) // NS, zblk, 0)
    plsc.subcore_barrier()

    iota16 = lax.iota(_i32, 16)

    def chunk(i, c):
        idx_s = src_v.at[i]
        idx_d = dst_v.at[i]
        cp1 = pltpu.async_copy(featx_hbm.at[idx_s], rows_v, sem1)
        cp2 = pltpu.async_copy(ert_hbm.at[idx_d], er_v, sem2)
        cp3 = pltpu.async_copy(ewr_hbm.at[wid, i], ew_v, sem3)
        cp1.wait()
        cp2.wait()
        cp3.wait()

        def group(g, c2):
            eids = g * 16 + iota16
            ewv = ew_v[pl.ds(g * 16, 16)]
            for h in range(H):
                colh = jnp.full((16,), 128 + h, _i32)
                el = plsc.load_gather(rows_v, [eids, colh])
                er = plsc.load_gather(er_v, [eids, jnp.full((16,), h, _i32)])
                e = el + er
                e = jnp.where(e > 0, e, NEG * e) * ewv
                plsc.store_scatter(rows_v, [eids, colh], jnp.exp(e))
            for t in range(16):
                row = g * 16 + t
                rowv = jnp.full((16,), row, _i32)
                spl = [plsc.load_gather(rows_v,
                                        [rowv, jnp.full((16,), 128 + h, _i32)])
                       for h in range(H)]
                for k in range(8):
                    seg = rows_v[row, pl.ds(k * 16, 16)]
                    rows_v[row, pl.ds(k * 16, 16)] = seg * spl[k // 2]
            return c2

        lax.fori_loop(0, G, group, 0)
        pltpu.sync_copy(rows_v, acc_sh.at[idx_d], add=True)
        return c

    lax.fori_loop(0, CH, chunk, 0)
    plsc.subcore_barrier()

    def dblk(i, c):
        j = sid + NS * i

        @pl.when(j < N // B)
        def _():
            off = pl.multiple_of(j * B, 8)
            pltpu.sync_copy(acc_sh.at[pl.ds(off, B)],
                            out_hbm.at[cid, pl.ds(off, B)])
        return c

    lax.fori_loop(0, (N // B + NS - 1) // NS, dblk, 0)


@functools.cache
def _edge_kernel_fn():
    return pl.kernel(
        _edge_body,
        out_type=jax.ShapeDtypeStruct((NC, N, FX), _f32),
        mesh=plsc.VectorSubcoreMesh(core_axis_name="c", subcore_axis_name="s"),
        compiler_params=pltpu.CompilerParams(use_tc_tiling_on_sc=False,
                                             needs_layout_passes=False),
        scratch_types=[
            pltpu.VMEM((CH, B), _i32),
            pltpu.VMEM((CH, B), _i32),
            pltpu.VMEM((B,), _f32),
            pltpu.VMEM((B, FX), _f32),
            pltpu.VMEM((B, ERT), _f32),
            pltpu.VMEM_SHARED((N, FX), _f32),
            pltpu.SemaphoreType.DMA,
            pltpu.SemaphoreType.DMA,
            pltpu.SemaphoreType.DMA,
        ],
    )


def _edge_kernel(*args):
    return _edge_kernel_fn()(*args)


def _attn_mat(a):
    m = a.reshape(H, D)
    cols = []
    for h in range(16):
        if h < H:
            col = jnp.zeros((H, D), _f32).at[h].set(m[h]).reshape(H * D)
        else:
            col = jnp.zeros((H * D,), _f32)
        cols.append(col)
    return jnp.stack(cols, axis=1)


def kernel(features, edge_index, edge_w, W1, al1, ar1, W2, al2, ar2,
           W_out, b_out):
    src_r = edge_index[0].reshape(NW, CH, B)
    dst_r = edge_index[1].reshape(NW, CH, B)
    ew_r = edge_w.reshape(NW, CH, B)

    a1el = _attn_mat(al1)
    a1er = _attn_mat(ar1)
    a2el = _attn_mat(al2)
    a2er = _attn_mat(ar2)
    e16 = jnp.concatenate(
        [jnp.kron(jnp.eye(H, dtype=_f32), jnp.ones((1, D), _f32)),
         jnp.zeros((16 - H, H * D), _f32)], axis=0)

    featx1, ert1 = _tc_layer1(features, W1.T, a1el, a1er)
    acc1 = _edge_kernel(featx1, ert1, src_r, dst_r, ew_r)
    featx2, ert2, rst1 = _tc_layer2(acc1, W2.T, a2el, a2er, e16)
    acc2 = _edge_kernel(featx2, ert2, src_r, dst_r, ew_r)
    return _tc_final(acc2, rst1, W_out[:, :OUT].T, W_out[:, OUT:].T,
                     e16, b_out.reshape(1, OUT))


# speedup vs baseline: 1.2160x; 1.2160x over previous
import functools

import jax
import jax.numpy as jnp
from jax import lax
from jax.experimental import pallas as pl
from jax.experimental.pallas import tpu as pltpu
from jax.experimental.pallas import tpu_sc as plsc

N = 10000
E = 320000
IN = 128
OUT = 128
H = 4
D = 32
NEG = 0.1

FX = 144
ERT = 16

NC = 2
NS = 16
NW = NC * NS
EW = E // NW
B = 80
CH = EW // B
G = B // 16
NPS = N // NS

_f32 = jnp.float32
_i32 = jnp.int32


def _dot(a, b):
    return jnp.dot(a, b, preferred_element_type=_f32,
                   precision=lax.Precision.HIGHEST)


def _layer1_body(x_ref, w1t_ref, ael_ref, aer_ref, featx_ref, ert_ref):
    f = _dot(x_ref[...], w1t_ref[...])
    elp = _dot(f, ael_ref[...])
    featx_ref[...] = jnp.concatenate([f, elp], axis=-1)
    ert_ref[...] = _dot(f, aer_ref[...])


def _layer2_body(acc_ref, w2t_ref, ael_ref, aer_ref, e16_ref,
                 featx_ref, ert_ref, rst1_ref):
    a = acc_ref[0] + acc_ref[1]
    den = _dot(a[:, 128:144], e16_ref[...]) + 1e-9
    r1 = jnp.maximum(a[:, :128] / den, 0.0)
    rst1_ref[...] = r1
    f = _dot(r1, w2t_ref[...])
    elp = _dot(f, ael_ref[...])
    featx_ref[...] = jnp.concatenate([f, elp], axis=-1)
    ert_ref[...] = _dot(f, aer_ref[...])


def _final_body(acc_ref, rst1_ref, wo1t_ref, wo2t_ref, e16_ref, b_ref,
                out_ref):
    a = acc_ref[0] + acc_ref[1]
    den = _dot(a[:, 128:144], e16_ref[...]) + 1e-9
    r2 = jnp.maximum(a[:, :128] / den, 0.0)
    out_ref[...] = (_dot(rst1_ref[...], wo1t_ref[...])
                    + _dot(r2, wo2t_ref[...]) + b_ref[...])


_R = 2000


def _tc_layer1(x, w1t, ael, aer):
    return pl.pallas_call(
        _layer1_body,
        grid=(N // _R,),
        in_specs=[
            pl.BlockSpec((_R, IN), lambda i: (i, 0)),
            pl.BlockSpec((IN, IN), lambda i: (0, 0)),
            pl.BlockSpec((IN, 16), lambda i: (0, 0)),
            pl.BlockSpec((IN, 16), lambda i: (0, 0)),
        ],
        out_specs=[
            pl.BlockSpec((_R, FX), lambda i: (i, 0)),
            pl.BlockSpec((_R, ERT), lambda i: (i, 0)),
        ],
        out_shape=[
            jax.ShapeDtypeStruct((N, FX), _f32),
            jax.ShapeDtypeStruct((N, ERT), _f32),
        ],
    )(x, w1t, ael, aer)


def _tc_layer2(acc, w2t, ael, aer, e16):
    return pl.pallas_call(
        _layer2_body,
        grid=(N // _R,),
        in_specs=[
            pl.BlockSpec((NC, _R, FX), lambda i: (0, i, 0)),
            pl.BlockSpec((IN, IN), lambda i: (0, 0)),
            pl.BlockSpec((IN, 16), lambda i: (0, 0)),
            pl.BlockSpec((IN, 16), lambda i: (0, 0)),
            pl.BlockSpec((16, IN), lambda i: (0, 0)),
        ],
        out_specs=[
            pl.BlockSpec((_R, FX), lambda i: (i, 0)),
            pl.BlockSpec((_R, ERT), lambda i: (i, 0)),
            pl.BlockSpec((_R, IN), lambda i: (i, 0)),
        ],
        out_shape=[
            jax.ShapeDtypeStruct((N, FX), _f32),
            jax.ShapeDtypeStruct((N, ERT), _f32),
            jax.ShapeDtypeStruct((N, IN), _f32),
        ],
    )(acc, w2t, ael, aer, e16)


def _tc_final(acc, rst1, wo1t, wo2t, e16, b):
    return pl.pallas_call(
        _final_body,
        grid=(N // _R,),
        in_specs=[
            pl.BlockSpec((NC, _R, FX), lambda i: (0, i, 0)),
            pl.BlockSpec((_R, IN), lambda i: (i, 0)),
            pl.BlockSpec((IN, OUT), lambda i: (0, 0)),
            pl.BlockSpec((IN, OUT), lambda i: (0, 0)),
            pl.BlockSpec((16, IN), lambda i: (0, 0)),
            pl.BlockSpec((1, OUT), lambda i: (0, 0)),
        ],
        out_specs=pl.BlockSpec((_R, OUT), lambda i: (i, 0)),
        out_shape=jax.ShapeDtypeStruct((N, OUT), _f32),
    )(acc, rst1, wo1t, wo2t, e16, b)


def _edge_body(featx_hbm, ert_hbm, srcr_hbm, dstr_hbm, ewr_hbm, out_hbm,
               idxs0, idxs1, idxd0, idxd1, ewb0, ewb1,
               rows0, rows1, erb0, erb1, acc_sh, isem0, isem1, gsem0, gsem1):
    cid = lax.axis_index("c")
    sid = lax.axis_index("s")
    wid = sid * NC + cid

    ibufs = [(idxs0, idxd0, ewb0, isem0), (idxs1, idxd1, ewb1, isem1)]
    gbufs = [(rows0, erb0, gsem0), (rows1, erb1, gsem1)]

    def load_idx(i, b):
        idxs, idxd, ewb, sem = ibufs[b]
        pltpu.async_copy(srcr_hbm.at[wid, i], idxs, sem)
        pltpu.async_copy(dstr_hbm.at[wid, i], idxd, sem)
        pltpu.async_copy(ewr_hbm.at[wid, i], ewb, sem)

    def wait_idx(i, b):
        idxs, idxd, ewb, sem = ibufs[b]
        pltpu.make_async_copy(srcr_hbm.at[wid, i], idxs, sem).wait()
        pltpu.make_async_copy(dstr_hbm.at[wid, i], idxd, sem).wait()
        pltpu.make_async_copy(ewr_hbm.at[wid, i], ewb, sem).wait()

    def gather(b):
        idxs, idxd, _, _ = ibufs[b]
        rows, erb, sem = gbufs[b]
        pltpu.async_copy(featx_hbm.at[idxs], rows, sem)
        pltpu.async_copy(ert_hbm.at[idxd], erb, sem)

    def wait_gather(b):
        idxs, idxd, _, _ = ibufs[b]
        rows, erb, sem = gbufs[b]
        pltpu.make_async_copy(featx_hbm.at[idxs], rows, sem).wait()
        pltpu.make_async_copy(ert_hbm.at[idxd], erb, sem).wait()

    iota16 = lax.iota(_i32, 16)

    def compute_scatter(b):
        _, idxd, ewb, _ = ibufs[b]
        rows, erb, _ = gbufs[b]

        def group(g, c2):
            eids = g * 16 + iota16
            ewv = ewb[pl.ds(g * 16, 16)]
            for h in range(H):
                colh = jnp.full((16,), 128 + h, _i32)
                el = plsc.load_gather(rows, [eids, colh])
                er = plsc.load_gather(erb, [eids, jnp.full((16,), h, _i32)])
                e = el + er
                e = jnp.where(e > 0, e, NEG * e) * ewv
                plsc.store_scatter(rows, [eids, colh], jnp.exp(e))
            for t in range(16):
                row = g * 16 + t
                rowv = jnp.full((16,), row, _i32)
                spl = [plsc.load_gather(rows,
                                        [rowv, jnp.full((16,), 128 + h, _i32)])
                       for h in range(H)]
                for k in range(8):
                    seg = rows[row, pl.ds(k * 16, 16)]
                    rows[row, pl.ds(k * 16, 16)] = seg * spl[k // 2]
            return c2

        lax.fori_loop(0, G, group, 0)
        pltpu.sync_copy(rows, acc_sh.at[idxd], add=True)

    zv = jnp.zeros((16,), _f32)

    def zrow(r, c):
        for k in range(FX // 16):
            rows0[r, pl.ds(k * 16, 16)] = zv
        return c

    lax.fori_loop(0, B, zrow, 0)

    def zblk(i, c):
        j = sid + NS * i

        @pl.when(j < N // B)
        def _():
            off = pl.multiple_of(j * B, 8)
            pltpu.sync_copy(rows0, acc_sh.at[pl.ds(off, B)])
        return c

    lax.fori_loop(0, (N // B + NS - 1) // NS, zblk, 0)
    plsc.subcore_barrier()

    pltpu.sync_copy(srcr_hbm.at[wid, 0], idxs0)
    pltpu.sync_copy(dstr_hbm.at[wid, 0], idxd0)
    pltpu.sync_copy(ewr_hbm.at[wid, 0], ewb0)
    gather(0)
    load_idx(1, 1)

    def pair(j, c):
        i = 2 * j
        wait_idx(i + 1, 1)
        gather(1)
        wait_gather(0)
        compute_scatter(0)
        load_idx(i + 2, 0)
        wait_idx(i + 2, 0)
        gather(0)
        wait_gather(1)
        compute_scatter(1)

        @pl.when(i + 3 < CH)
        def _():
            load_idx(i + 3, 1)
        return c

    lax.fori_loop(0, CH // 2, pair, 0)
    wait_gather(0)
    compute_scatter(0)
    plsc.subcore_barrier()

    def dblk(i, c):
        j = sid + NS * i

        @pl.when(j < N // B)
        def _():
            off = pl.multiple_of(j * B, 8)
            pltpu.sync_copy(acc_sh.at[pl.ds(off, B)],
                            out_hbm.at[cid, pl.ds(off, B)])
        return c

    lax.fori_loop(0, (N // B + NS - 1) // NS, dblk, 0)


@functools.cache
def _edge_kernel_fn():
    return pl.kernel(
        _edge_body,
        out_type=jax.ShapeDtypeStruct((NC, N, FX), _f32),
        mesh=plsc.VectorSubcoreMesh(core_axis_name="c", subcore_axis_name="s"),
        compiler_params=pltpu.CompilerParams(use_tc_tiling_on_sc=False,
                                             needs_layout_passes=False),
        scratch_types=[
            pltpu.VMEM((B,), _i32),
            pltpu.VMEM((B,), _i32),
            pltpu.VMEM((B,), _i32),
            pltpu.VMEM((B,), _i32),
            pltpu.VMEM((B,), _f32),
            pltpu.VMEM((B,), _f32),
            pltpu.VMEM((B, FX), _f32),
            pltpu.VMEM((B, FX), _f32),
            pltpu.VMEM((B, ERT), _f32),
            pltpu.VMEM((B, ERT), _f32),
            pltpu.VMEM_SHARED((N, FX), _f32),
            pltpu.SemaphoreType.DMA,
            pltpu.SemaphoreType.DMA,
            pltpu.SemaphoreType.DMA,
            pltpu.SemaphoreType.DMA,
        ],
    )


def _edge_kernel(*args):
    return _edge_kernel_fn()(*args)


def _attn_mat(a):
    m = a.reshape(H, D)
    cols = []
    for h in range(16):
        if h < H:
            col = jnp.zeros((H, D), _f32).at[h].set(m[h]).reshape(H * D)
        else:
            col = jnp.zeros((H * D,), _f32)
        cols.append(col)
    return jnp.stack(cols, axis=1)


def kernel(features, edge_index, edge_w, W1, al1, ar1, W2, al2, ar2,
           W_out, b_out):
    src_r = edge_index[0].reshape(NW, CH, B)
    dst_r = edge_index[1].reshape(NW, CH, B)
    ew_r = edge_w.reshape(NW, CH, B)

    a1el = _attn_mat(al1)
    a1er = _attn_mat(ar1)
    a2el = _attn_mat(al2)
    a2er = _attn_mat(ar2)
    e16 = jnp.concatenate(
        [jnp.kron(jnp.eye(H, dtype=_f32), jnp.ones((1, D), _f32)),
         jnp.zeros((16 - H, H * D), _f32)], axis=0)

    featx1, ert1 = _tc_layer1(features, W1.T, a1el, a1er)
    acc1 = _edge_kernel(featx1, ert1, src_r, dst_r, ew_r)
    featx2, ert2, rst1 = _tc_layer2(acc1, W2.T, a2el, a2er, e16)
    acc2 = _edge_kernel(featx2, ert2, src_r, dst_r, ew_r)
    return _tc_final(acc2, rst1, W_out[:, :OUT].T, W_out[:, OUT:].T,
                     e16, b_out.reshape(1, OUT))


# speedup vs baseline: 1.4213x; 1.1689x over previous
import functools

import jax
import jax.numpy as jnp
from jax import lax
from jax.experimental import pallas as pl
from jax.experimental.pallas import tpu as pltpu
from jax.experimental.pallas import tpu_sc as plsc

N = 10000
E = 320000
IN = 128
OUT = 128
H = 4
D = 32
NEG = 0.1

FX = 144
ERT = 16

NC = 2
NS = 16
NW = NC * NS
EW = E // NW
B = 80
CH = EW // B
G = B // 16
NPS = N // NS

_f32 = jnp.float32
_i32 = jnp.int32


def _dot(a, b):
    return jnp.dot(a, b, preferred_element_type=_f32,
                   precision=lax.Precision.HIGHEST)


def _layer1_body(x_ref, w1t_ref, ael_ref, aer_ref, featx_ref, ert_ref):
    f = _dot(x_ref[...], w1t_ref[...])
    elp = _dot(f, ael_ref[...])
    featx_ref[...] = jnp.concatenate([f, elp], axis=-1)
    ert_ref[...] = _dot(f, aer_ref[...])


def _layer2_body(acc_ref, w2t_ref, ael_ref, aer_ref, e16_ref,
                 featx_ref, ert_ref, rst1_ref):
    a = acc_ref[0] + acc_ref[1]
    den = _dot(a[:, 128:144], e16_ref[...]) + 1e-9
    r1 = jnp.maximum(a[:, :128] / den, 0.0)
    rst1_ref[...] = r1
    f = _dot(r1, w2t_ref[...])
    elp = _dot(f, ael_ref[...])
    featx_ref[...] = jnp.concatenate([f, elp], axis=-1)
    ert_ref[...] = _dot(f, aer_ref[...])


def _final_body(acc_ref, rst1_ref, wo1t_ref, wo2t_ref, e16_ref, b_ref,
                out_ref):
    a = acc_ref[0] + acc_ref[1]
    den = _dot(a[:, 128:144], e16_ref[...]) + 1e-9
    r2 = jnp.maximum(a[:, :128] / den, 0.0)
    out_ref[...] = (_dot(rst1_ref[...], wo1t_ref[...])
                    + _dot(r2, wo2t_ref[...]) + b_ref[...])


_R = 2000


def _tc_layer1(x, w1t, ael, aer):
    return pl.pallas_call(
        _layer1_body,
        grid=(N // _R,),
        in_specs=[
            pl.BlockSpec((_R, IN), lambda i: (i, 0)),
            pl.BlockSpec((IN, IN), lambda i: (0, 0)),
            pl.BlockSpec((IN, 16), lambda i: (0, 0)),
            pl.BlockSpec((IN, 16), lambda i: (0, 0)),
        ],
        out_specs=[
            pl.BlockSpec((_R, FX), lambda i: (i, 0)),
            pl.BlockSpec((_R, ERT), lambda i: (i, 0)),
        ],
        out_shape=[
            jax.ShapeDtypeStruct((N, FX), _f32),
            jax.ShapeDtypeStruct((N, ERT), _f32),
        ],
    )(x, w1t, ael, aer)


def _tc_layer2(acc, w2t, ael, aer, e16):
    return pl.pallas_call(
        _layer2_body,
        grid=(N // _R,),
        in_specs=[
            pl.BlockSpec((NC, _R, FX), lambda i: (0, i, 0)),
            pl.BlockSpec((IN, IN), lambda i: (0, 0)),
            pl.BlockSpec((IN, 16), lambda i: (0, 0)),
            pl.BlockSpec((IN, 16), lambda i: (0, 0)),
            pl.BlockSpec((16, IN), lambda i: (0, 0)),
        ],
        out_specs=[
            pl.BlockSpec((_R, FX), lambda i: (i, 0)),
            pl.BlockSpec((_R, ERT), lambda i: (i, 0)),
            pl.BlockSpec((_R, IN), lambda i: (i, 0)),
        ],
        out_shape=[
            jax.ShapeDtypeStruct((N, FX), _f32),
            jax.ShapeDtypeStruct((N, ERT), _f32),
            jax.ShapeDtypeStruct((N, IN), _f32),
        ],
    )(acc, w2t, ael, aer, e16)


def _tc_final(acc, rst1, wo1t, wo2t, e16, b):
    return pl.pallas_call(
        _final_body,
        grid=(N // _R,),
        in_specs=[
            pl.BlockSpec((NC, _R, FX), lambda i: (0, i, 0)),
            pl.BlockSpec((_R, IN), lambda i: (i, 0)),
            pl.BlockSpec((IN, OUT), lambda i: (0, 0)),
            pl.BlockSpec((IN, OUT), lambda i: (0, 0)),
            pl.BlockSpec((16, IN), lambda i: (0, 0)),
            pl.BlockSpec((1, OUT), lambda i: (0, 0)),
        ],
        out_specs=pl.BlockSpec((_R, OUT), lambda i: (i, 0)),
        out_shape=jax.ShapeDtypeStruct((N, OUT), _f32),
    )(acc, rst1, wo1t, wo2t, e16, b)


def _edge_body(featx_hbm, ert_hbm, srcr_hbm, dstr_hbm, ewr_hbm, out_hbm,
               idxs0, idxs1, ewb0, ewb1, dst_v,
               rows0, rows1, erb0, erb1, acc_sh,
               isem0, isem1, gsem0, gsem1, ssem0, ssem1):
    cid = lax.axis_index("c")
    sid = lax.axis_index("s")
    wid = sid * NC + cid

    ibufs = [(idxs0, ewb0, isem0), (idxs1, ewb1, isem1)]
    gbufs = [(rows0, erb0, gsem0, ssem0), (rows1, erb1, gsem1, ssem1)]

    def load_idx(i, b):
        idxs, ewb, sem = ibufs[b]
        pltpu.async_copy(srcr_hbm.at[wid, i], idxs, sem)
        pltpu.async_copy(ewr_hbm.at[wid, i], ewb, sem)

    def wait_idx(i, b):
        idxs, ewb, sem = ibufs[b]
        pltpu.make_async_copy(srcr_hbm.at[wid, i], idxs, sem).wait()
        pltpu.make_async_copy(ewr_hbm.at[wid, i], ewb, sem).wait()

    def gather(i, b):
        idxs, _, _ = ibufs[b]
        rows, erb, sem, _ = gbufs[b]
        pltpu.async_copy(featx_hbm.at[idxs], rows, sem)
        pltpu.async_copy(ert_hbm.at[dst_v.at[i]], erb, sem)

    def wait_gather(i, b):
        idxs, _, _ = ibufs[b]
        rows, erb, sem, _ = gbufs[b]
        pltpu.make_async_copy(featx_hbm.at[idxs], rows, sem).wait()
        pltpu.make_async_copy(ert_hbm.at[dst_v.at[i]], erb, sem).wait()

    def scatter(i, b):
        rows, _, _, sem = gbufs[b]
        pltpu.async_copy(rows, acc_sh.at[dst_v.at[i]], sem, add=True)

    def wait_scatter(i, b):
        rows, _, _, sem = gbufs[b]
        pltpu.make_async_copy(rows, acc_sh.at[dst_v.at[i]], sem).wait()

    iota16 = lax.iota(_i32, 16)

    def compute(b):
        _, ewb, _ = ibufs[b]
        rows, erb, _, _ = gbufs[b]

        def group(g, c2):
            eids = g * 16 + iota16
            ewv = ewb[pl.ds(g * 16, 16)]
            for h in range(H):
                colh = jnp.full((16,), 128 + h, _i32)
                el = plsc.load_gather(rows, [eids, colh])
                er = plsc.load_gather(erb, [eids, jnp.full((16,), h, _i32)])
                e = el + er
                e = jnp.where(e > 0, e, NEG * e) * ewv
                plsc.store_scatter(rows, [eids, colh], jnp.exp(e))
            for t in range(16):
                row = g * 16 + t
                rowv = jnp.full((16,), row, _i32)
                spl = [plsc.load_gather(rows,
                                        [rowv, jnp.full((16,), 128 + h, _i32)])
                       for h in range(H)]
                for k in range(8):
                    seg = rows[row, pl.ds(k * 16, 16)]
                    rows[row, pl.ds(k * 16, 16)] = seg * spl[k // 2]
            return c2

        lax.fori_loop(0, G, group, 0)

    zv = jnp.zeros((16,), _f32)

    def zrow(r, c):
        for k in range(FX // 16):
            rows0[r, pl.ds(k * 16, 16)] = zv
        return c

    lax.fori_loop(0, B, zrow, 0)

    def zblk(i, c):
        j = sid + NS * i

        @pl.when(j < N // B)
        def _():
            off = pl.multiple_of(j * B, 8)
            pltpu.sync_copy(rows0, acc_sh.at[pl.ds(off, B)])
        return c

    lax.fori_loop(0, (N // B + NS - 1) // NS, zblk, 0)
    plsc.subcore_barrier()

    pltpu.sync_copy(dstr_hbm.at[wid], dst_v)
    pltpu.sync_copy(srcr_hbm.at[wid, 0], idxs0)
    pltpu.sync_copy(ewr_hbm.at[wid, 0], ewb0)
    gather(0, 0)
    load_idx(1, 1)

    def pair(j, c):
        i = 2 * j
        wait_idx(i + 1, 1)

        @pl.when(i >= 1)
        def _():
            wait_scatter(i - 1, 1)
        gather(i + 1, 1)
        wait_gather(i, 0)
        compute(0)
        scatter(i, 0)
        load_idx(i + 2, 0)
        wait_idx(i + 2, 0)
        wait_scatter(i, 0)
        gather(i + 2, 0)
        wait_gather(i + 1, 1)
        compute(1)
        scatter(i + 1, 1)

        @pl.when(i + 3 < CH)
        def _():
            load_idx(i + 3, 1)
        return c

    lax.fori_loop(0, CH // 2, pair, 0)
    wait_gather(CH - 1, 0)
    wait_scatter(CH - 2, 1)
    compute(0)
    scatter(CH - 1, 0)
    wait_scatter(CH - 1, 0)
    plsc.subcore_barrier()

    def dblk(i, c):
        j = sid + NS * i

        @pl.when(j < N // B)
        def _():
            off = pl.multiple_of(j * B, 8)
            pltpu.sync_copy(acc_sh.at[pl.ds(off, B)],
                            out_hbm.at[cid, pl.ds(off, B)])
        return c

    lax.fori_loop(0, (N // B + NS - 1) // NS, dblk, 0)


@functools.cache
def _edge_kernel_fn():
    return pl.kernel(
        _edge_body,
        out_type=jax.ShapeDtypeStruct((NC, N, FX), _f32),
        mesh=plsc.VectorSubcoreMesh(core_axis_name="c", subcore_axis_name="s"),
        compiler_params=pltpu.CompilerParams(use_tc_tiling_on_sc=False,
                                             needs_layout_passes=False),
        scratch_types=[
            pltpu.VMEM((B,), _i32),
            pltpu.VMEM((B,), _i32),
            pltpu.VMEM((B,), _f32),
            pltpu.VMEM((B,), _f32),
            pltpu.VMEM((CH, B), _i32),
            pltpu.VMEM((B, FX), _f32),
            pltpu.VMEM((B, FX), _f32),
            pltpu.VMEM((B, ERT), _f32),
            pltpu.VMEM((B, ERT), _f32),
            pltpu.VMEM_SHARED((N, FX), _f32),
            pltpu.SemaphoreType.DMA,
            pltpu.SemaphoreType.DMA,
            pltpu.SemaphoreType.DMA,
            pltpu.SemaphoreType.DMA,
            pltpu.SemaphoreType.DMA,
            pltpu.SemaphoreType.DMA,
        ],
    )


def _edge_kernel(*args):
    return _edge_kernel_fn()(*args)


def _attn_mat(a):
    m = a.reshape(H, D)
    cols = []
    for h in range(16):
        if h < H:
            col = jnp.zeros((H, D), _f32).at[h].set(m[h]).reshape(H * D)
        else:
            col = jnp.zeros((H * D,), _f32)
        cols.append(col)
    return jnp.stack(cols, axis=1)


def kernel(features, edge_index, edge_w, W1, al1, ar1, W2, al2, ar2,
           W_out, b_out):
    src_r = edge_index[0].reshape(NW, CH, B)
    dst_r = edge_index[1].reshape(NW, CH, B)
    ew_r = edge_w.reshape(NW, CH, B)

    a1el = _attn_mat(al1)
    a1er = _attn_mat(ar1)
    a2el = _attn_mat(al2)
    a2er = _attn_mat(ar2)
    e16 = jnp.concatenate(
        [jnp.kron(jnp.eye(H, dtype=_f32), jnp.ones((1, D), _f32)),
         jnp.zeros((16 - H, H * D), _f32)], axis=0)

    featx1, ert1 = _tc_layer1(features, W1.T, a1el, a1er)
    acc1 = _edge_kernel(featx1, ert1, src_r, dst_r, ew_r)
    featx2, ert2, rst1 = _tc_layer2(acc1, W2.T, a2el, a2er, e16)
    acc2 = _edge_kernel(featx2, ert2, src_r, dst_r, ew_r)
    return _tc_final(acc2, rst1, W_out[:, :OUT].T, W_out[:, OUT:].T,
                     e16, b_out.reshape(1, OUT))


# speedup vs baseline: 1.7314x; 1.2182x over previous
import functools

import jax
import jax.numpy as jnp
from jax import lax
from jax.experimental import pallas as pl
from jax.experimental.pallas import tpu as pltpu
from jax.experimental.pallas import tpu_sc as plsc

N = 10000
E = 320000
IN = 128
OUT = 128
H = 4
D = 32
NEG = 0.1

FX = 144
ERT = 16

NC = 2
NS = 16
NW = NC * NS
EW = E // NW
B = 80
CH = EW // B
G = B // 16
NPS = N // NS

_f32 = jnp.float32
_i32 = jnp.int32


def _dot(a, b):
    return jnp.dot(a, b, preferred_element_type=_f32,
                   precision=lax.Precision.HIGHEST)


def _layer1_body(x_ref, w1t_ref, ael_ref, aer_ref, featx_ref, ert_ref):
    f = _dot(x_ref[...], w1t_ref[...])
    elp = _dot(f, ael_ref[...])
    featx_ref[...] = jnp.concatenate([f, elp], axis=-1)
    ert_ref[...] = _dot(f, aer_ref[...])


def _layer2_body(acc_ref, w2t_ref, ael_ref, aer_ref, e16_ref,
                 featx_ref, ert_ref, rst1_ref):
    a = acc_ref[0] + acc_ref[1]
    den = _dot(a[:, 128:144], e16_ref[...]) + 1e-9
    r1 = jnp.maximum(a[:, :128] / den, 0.0)
    rst1_ref[...] = r1
    f = _dot(r1, w2t_ref[...])
    elp = _dot(f, ael_ref[...])
    featx_ref[...] = jnp.concatenate([f, elp], axis=-1)
    ert_ref[...] = _dot(f, aer_ref[...])


def _final_body(acc_ref, rst1_ref, wo1t_ref, wo2t_ref, e16_ref, b_ref,
                out_ref):
    a = acc_ref[0] + acc_ref[1]
    den = _dot(a[:, 128:144], e16_ref[...]) + 1e-9
    r2 = jnp.maximum(a[:, :128] / den, 0.0)
    out_ref[...] = (_dot(rst1_ref[...], wo1t_ref[...])
                    + _dot(r2, wo2t_ref[...]) + b_ref[...])


_R = 2000


def _tc_layer1(x, w1t, ael, aer):
    return pl.pallas_call(
        _layer1_body,
        grid=(N // _R,),
        in_specs=[
            pl.BlockSpec((_R, IN), lambda i: (i, 0)),
            pl.BlockSpec((IN, IN), lambda i: (0, 0)),
            pl.BlockSpec((IN, 16), lambda i: (0, 0)),
            pl.BlockSpec((IN, 16), lambda i: (0, 0)),
        ],
        out_specs=[
            pl.BlockSpec((_R, FX), lambda i: (i, 0)),
            pl.BlockSpec((_R, ERT), lambda i: (i, 0)),
        ],
        out_shape=[
            jax.ShapeDtypeStruct((N, FX), _f32),
            jax.ShapeDtypeStruct((N, ERT), _f32),
        ],
    )(x, w1t, ael, aer)


def _tc_layer2(acc, w2t, ael, aer, e16):
    return pl.pallas_call(
        _layer2_body,
        grid=(N // _R,),
        in_specs=[
            pl.BlockSpec((NC, _R, FX), lambda i: (0, i, 0)),
            pl.BlockSpec((IN, IN), lambda i: (0, 0)),
            pl.BlockSpec((IN, 16), lambda i: (0, 0)),
            pl.BlockSpec((IN, 16), lambda i: (0, 0)),
            pl.BlockSpec((16, IN), lambda i: (0, 0)),
        ],
        out_specs=[
            pl.BlockSpec((_R, FX), lambda i: (i, 0)),
            pl.BlockSpec((_R, ERT), lambda i: (i, 0)),
            pl.BlockSpec((_R, IN), lambda i: (i, 0)),
        ],
        out_shape=[
            jax.ShapeDtypeStruct((N, FX), _f32),
            jax.ShapeDtypeStruct((N, ERT), _f32),
            jax.ShapeDtypeStruct((N, IN), _f32),
        ],
    )(acc, w2t, ael, aer, e16)


def _tc_final(acc, rst1, wo1t, wo2t, e16, b):
    return pl.pallas_call(
        _final_body,
        grid=(N // _R,),
        in_specs=[
            pl.BlockSpec((NC, _R, FX), lambda i: (0, i, 0)),
            pl.BlockSpec((_R, IN), lambda i: (i, 0)),
            pl.BlockSpec((IN, OUT), lambda i: (0, 0)),
            pl.BlockSpec((IN, OUT), lambda i: (0, 0)),
            pl.BlockSpec((16, IN), lambda i: (0, 0)),
            pl.BlockSpec((1, OUT), lambda i: (0, 0)),
        ],
        out_specs=pl.BlockSpec((_R, OUT), lambda i: (i, 0)),
        out_shape=jax.ShapeDtypeStruct((N, OUT), _f32),
    )(acc, rst1, wo1t, wo2t, e16, b)


def _edge_body(featx_hbm, ert_hbm, srcr_hbm, dstr_hbm, ewr_hbm, out_hbm,
               idxs0, idxs1, ewb0, ewb1, dst_v,
               rows0, rows1, erb0, erb1, acc_sh,
               isem0, isem1, gsem0, gsem1, ssem0, ssem1):
    cid = lax.axis_index("c")
    sid = lax.axis_index("s")
    wid = sid * NC + cid

    ibufs = [(idxs0, ewb0, isem0), (idxs1, ewb1, isem1)]
    gbufs = [(rows0, erb0, gsem0, ssem0), (rows1, erb1, gsem1, ssem1)]

    def load_idx(i, b):
        idxs, ewb, sem = ibufs[b]
        pltpu.async_copy(srcr_hbm.at[wid, i], idxs, sem)
        pltpu.async_copy(ewr_hbm.at[wid, i], ewb, sem)

    def wait_idx(i, b):
        idxs, ewb, sem = ibufs[b]
        pltpu.make_async_copy(srcr_hbm.at[wid, i], idxs, sem).wait()
        pltpu.make_async_copy(ewr_hbm.at[wid, i], ewb, sem).wait()

    def gather(i, b):
        idxs, _, _ = ibufs[b]
        rows, erb, sem, _ = gbufs[b]
        pltpu.async_copy(featx_hbm.at[idxs], rows, sem)
        pltpu.async_copy(ert_hbm.at[dst_v.at[i]], erb, sem)

    def wait_gather(i, b):
        idxs, _, _ = ibufs[b]
        rows, erb, sem, _ = gbufs[b]
        pltpu.make_async_copy(featx_hbm.at[idxs], rows, sem).wait()
        pltpu.make_async_copy(ert_hbm.at[dst_v.at[i]], erb, sem).wait()

    def scatter(i, b):
        rows, _, _, sem = gbufs[b]
        pltpu.async_copy(rows, acc_sh.at[dst_v.at[i]], sem, add=True)

    def wait_scatter(i, b):
        rows, _, _, sem = gbufs[b]
        pltpu.make_async_copy(rows, acc_sh.at[dst_v.at[i]], sem).wait()

    iota16 = lax.iota(_i32, 16)

    def compute(b):
        _, ewb, _ = ibufs[b]
        rows, erb, _, _ = gbufs[b]

        def group(g, c2):
            eids = g * 16 + iota16
            ewv = ewb[pl.ds(g * 16, 16)]
            ees = []
            for h in range(H):
                colh = jnp.full((16,), 128 + h, _i32)
                el = plsc.load_gather(rows, [eids, colh])
                er = plsc.load_gather(erb, [eids, jnp.full((16,), h, _i32)])
                e = el + er
                e = jnp.where(e > 0, e, NEG * e) * ewv
                ee = jnp.exp(e)
                plsc.store_scatter(rows, [eids, colh], ee)
                ees.append(ee)
            for t in range(16):
                row = g * 16 + t
                tv = jnp.full((16,), t, _i32)
                spl = [lax.gather(
                    ees[h], tv.reshape(16, 1),
                    dimension_numbers=lax.GatherDimensionNumbers(
                        offset_dims=(), collapsed_slice_dims=(0,),
                        start_index_map=(0,)),
                    slice_sizes=(1,),
                    mode=lax.GatherScatterMode.PROMISE_IN_BOUNDS)
                    for h in range(H)]
                for k in range(8):
                    seg = rows[row, pl.ds(k * 16, 16)]
                    rows[row, pl.ds(k * 16, 16)] = seg * spl[k // 2]
            return c2

        lax.fori_loop(0, G, group, 0)

    zv = jnp.zeros((16,), _f32)

    def zrow(r, c):
        for k in range(FX // 16):
            rows0[r, pl.ds(k * 16, 16)] = zv
        return c

    lax.fori_loop(0, B, zrow, 0)

    def zblk(i, c):
        j = sid + NS * i

        @pl.when(j < N // B)
        def _():
            off = pl.multiple_of(j * B, 8)
            pltpu.sync_copy(rows0, acc_sh.at[pl.ds(off, B)])
        return c

    lax.fori_loop(0, (N // B + NS - 1) // NS, zblk, 0)
    plsc.subcore_barrier()

    pltpu.sync_copy(dstr_hbm.at[wid], dst_v)
    pltpu.sync_copy(srcr_hbm.at[wid, 0], idxs0)
    pltpu.sync_copy(ewr_hbm.at[wid, 0], ewb0)
    gather(0, 0)
    load_idx(1, 1)

    def pair(j, c):
        i = 2 * j
        wait_idx(i + 1, 1)

        @pl.when(i >= 1)
        def _():
            wait_scatter(i - 1, 1)
        gather(i + 1, 1)
        wait_gather(i, 0)
        compute(0)
        scatter(i, 0)
        load_idx(i + 2, 0)
        wait_idx(i + 2, 0)
        wait_scatter(i, 0)
        gather(i + 2, 0)
        wait_gather(i + 1, 1)
        compute(1)
        scatter(i + 1, 1)

        @pl.when(i + 3 < CH)
        def _():
            load_idx(i + 3, 1)
        return c

    lax.fori_loop(0, CH // 2, pair, 0)
    wait_gather(CH - 1, 0)
    wait_scatter(CH - 2, 1)
    compute(0)
    scatter(CH - 1, 0)
    wait_scatter(CH - 1, 0)
    plsc.subcore_barrier()

    def dblk(i, c):
        j = sid + NS * i

        @pl.when(j < N // B)
        def _():
            off = pl.multiple_of(j * B, 8)
            pltpu.sync_copy(acc_sh.at[pl.ds(off, B)],
                            out_hbm.at[cid, pl.ds(off, B)])
        return c

    lax.fori_loop(0, (N // B + NS - 1) // NS, dblk, 0)


@functools.cache
def _edge_kernel_fn():
    return pl.kernel(
        _edge_body,
        out_type=jax.ShapeDtypeStruct((NC, N, FX), _f32),
        mesh=plsc.VectorSubcoreMesh(core_axis_name="c", subcore_axis_name="s"),
        compiler_params=pltpu.CompilerParams(use_tc_tiling_on_sc=False,
                                             needs_layout_passes=False),
        scratch_types=[
            pltpu.VMEM((B,), _i32),
            pltpu.VMEM((B,), _i32),
            pltpu.VMEM((B,), _f32),
            pltpu.VMEM((B,), _f32),
            pltpu.VMEM((CH, B), _i32),
            pltpu.VMEM((B, FX), _f32),
            pltpu.VMEM((B, FX), _f32),
            pltpu.VMEM((B, ERT), _f32),
            pltpu.VMEM((B, ERT), _f32),
            pltpu.VMEM_SHARED((N, FX), _f32),
            pltpu.SemaphoreType.DMA,
            pltpu.SemaphoreType.DMA,
            pltpu.SemaphoreType.DMA,
            pltpu.SemaphoreType.DMA,
            pltpu.SemaphoreType.DMA,
            pltpu.SemaphoreType.DMA,
        ],
    )


def _edge_kernel(*args):
    return _edge_kernel_fn()(*args)


def _attn_mat(a):
    m = a.reshape(H, D)
    cols = []
    for h in range(16):
        if h < H:
            col = jnp.zeros((H, D), _f32).at[h].set(m[h]).reshape(H * D)
        else:
            col = jnp.zeros((H * D,), _f32)
        cols.append(col)
    return jnp.stack(cols, axis=1)


def kernel(features, edge_index, edge_w, W1, al1, ar1, W2, al2, ar2,
           W_out, b_out):
    src_r = edge_index[0].reshape(NW, CH, B)
    dst_r = edge_index[1].reshape(NW, CH, B)
    ew_r = edge_w.reshape(NW, CH, B)

    a1el = _attn_mat(al1)
    a1er = _attn_mat(ar1)
    a2el = _attn_mat(al2)
    a2er = _attn_mat(ar2)
    e16 = jnp.concatenate(
        [jnp.kron(jnp.eye(H, dtype=_f32), jnp.ones((1, D), _f32)),
         jnp.zeros((16 - H, H * D), _f32)], axis=0)

    featx1, ert1 = _tc_layer1(features, W1.T, a1el, a1er)
    acc1 = _edge_kernel(featx1, ert1, src_r, dst_r, ew_r)
    featx2, ert2, rst1 = _tc_layer2(acc1, W2.T, a2el, a2er, e16)
    acc2 = _edge_kernel(featx2, ert2, src_r, dst_r, ew_r)
    return _tc_final(acc2, rst1, W_out[:, :OUT].T, W_out[:, OUT:].T,
                     e16, b_out.reshape(1, OUT))
